# baseline (speedup 1.0000x reference)
import functools

import jax
import jax.numpy as jnp
from jax import lax
from jax.experimental import pallas as pl
from jax.experimental.pallas import tpu as pltpu
from jax.experimental.pallas import tpu_sc as plsc

B = 128
V = 100000
K = 50
KRAW = 64
OUT = 80
CAP = 2048
UNR = 25
BLK = 25
NW = 32
RPW = B // NW
TEMP = 0.8
P_TOP = 0.95
NEG = float("-inf")
IMAX = 0x7FFFFFFF


def _sc_extract_body(x_hbm, ov_hbm, oi_hbm, data_v, vals_c, idx_c, u_c,
                     stage_v, stage_i):
    wid = lax.axis_index("s") * 2 + lax.axis_index("c")
    iota16 = lax.iota(jnp.int32, 16)
    neg16 = jnp.full((16,), NEG, jnp.float32)

    def do_row(r, _):
        row = wid * RPW + r
        pltpu.sync_copy(x_hbm.at[pl.ds(row * V, V)], data_v)

    lane15 = jnp.full((16,), 15, jnp.int32)

    def slow_scan(cnt, base, t):
        cv = jnp.zeros((16,), jnp.int32) + jnp.minimum(cnt, CAP - 16)
        for w in range(BLK):
            b2 = base + w * 16
            v = data_v[pl.ds(b2, 16)]
            m = v >= t
            mi = m.astype(jnp.int32)
            pos = plsc.cumsum(mi)
            tgt = jnp.minimum(cv + pos - mi, CAP - 1)
            plsc.store_scatter(vals_c, [tgt], v, mask=m)
            plsc.store_scatter(idx_c, [tgt], iota16 + b2, mask=m)
            cv = cv + jnp.take(pos, lane15)
        return cnt

    def do_row(r, _):
        row = wid * RPW + r
        pltpu.sync_copy(x_hbm.at[pl.ds(row * V, V)], data_v)

        def top4_body(g, t):
            t1, t2, t3, t4 = t
            x = data_v[pl.ds(g * (16 * UNR), 16)]
            for w in range(1, UNR):
                x = jnp.maximum(x, data_v[pl.ds(g * (16 * UNR) + w * 16, 16)])
            t4 = jnp.maximum(t4, jnp.minimum(x, t3))
            t3 = jnp.maximum(t3, jnp.minimum(x, t2))
            t2 = jnp.maximum(t2, jnp.minimum(x, t1))
            t1 = jnp.maximum(t1, x)
            return t1, t2, t3, t4

        _, _, _, t4 = lax.fori_loop(0, V // (16 * UNR), top4_body,
                                    (neg16, neg16, neg16, neg16))
        t = jnp.min(t4)

        def blk_body(b, carry):
            cnt, psum, pbase = carry

            def run_slow(c, pbase=pbase, psum=psum):
                slow_scan(c, pbase, t)
                return c + psum

            cnt = lax.cond(psum > 0, run_slow, lambda c: c, cnt)
            base = b * BLK * 16
            acc = (data_v[pl.ds(base, 16)] >= t).astype(jnp.int32)
            for w in range(1, BLK):
                acc = acc + (data_v[pl.ds(base + w * 16, 16)] >=
                             t).astype(jnp.int32)
            nhit = jnp.sum(acc)
            return cnt, nhit, base

        cnt, psum, pbase = lax.fori_loop(
            0, V // (16 * BLK), blk_body,
            (jnp.int32(0), jnp.int32(0), jnp.int32(0)))

        def run_slow_tail(c):
            slow_scan(c, pbase, t)
            return c + psum

        cnt = lax.cond(psum > 0, run_slow_tail, lambda c: c, cnt)
        cnt = jnp.minimum(cnt, jnp.int32(CAP))
        nv = (cnt + 15) // 16

        def mono_body(j, _):
            x = vals_c[pl.ds(j * 16, 16)]
            b = plsc.bitcast(x + jnp.float32(0.0), jnp.int32)
            u = b ^ (lax.shift_right_arithmetic(b, 31) & jnp.int32(IMAX))
            valid = (j * 16 + iota16) < cnt
            u_c[pl.ds(j * 16, 16)] = jnp.where(valid, u,
                                               jnp.int32(-IMAX - 1))
            return 0

        lax.fori_loop(0, nv, mono_body, 0)

        tb = jnp.int32(0)
        for bit in range(31, -1, -1):
            cand = tb | (jnp.int32(1) << bit)
            probe = cand ^ jnp.int32(-IMAX - 1)

            def cnt_body(j, c, probe=probe):
                u = u_c[pl.ds(j * 16, 16)]
                return c + jnp.sum((u >= probe).astype(jnp.int32))

            c = lax.fori_loop(0, nv, cnt_body, jnp.int32(0))
            tb = jnp.where(c >= KRAW, cand, tb)
        t64 = tb ^ jnp.int32(-IMAX - 1)

        for jj in range(OUT // 16):
            stage_v[pl.ds(jj * 16, 16)] = neg16
            stage_i[pl.ds(jj * 16, 16)] = jnp.full((16,), IMAX, jnp.int32)

        def fcompact(j, oc):
            x = vals_c[pl.ds(j * 16, 16)]
            ix = idx_c[pl.ds(j * 16, 16)]
            u = u_c[pl.ds(j * 16, 16)]
            m = u >= t64
            s = jnp.sum(m.astype(jnp.int32))

            def do_store():
                plsc.store_compressed(stage_v.at[pl.ds(oc, 16)], x, mask=m)
                plsc.store_compressed(stage_i.at[pl.ds(oc, 16)], ix, mask=m)

            pl.when(oc + s <= OUT)(do_store)
            return oc + s

        lax.fori_loop(0, nv, fcompact, jnp.int32(0))

        pltpu.sync_copy(stage_v, ov_hbm.at[pl.ds(row * OUT, OUT)])
        pltpu.sync_copy(stage_i, oi_hbm.at[pl.ds(row * OUT, OUT)])
        return 0

    lax.fori_loop(0, RPW, do_row, 0)


@jax.jit
def _sc_extract(flat_logits):
    mesh = plsc.VectorSubcoreMesh(core_axis_name="c", subcore_axis_name="s")
    run = pl.kernel(
        _sc_extract_body,
        out_type=[
            jax.ShapeDtypeStruct((B * OUT,), jnp.float32),
            jax.ShapeDtypeStruct((B * OUT,), jnp.int32),
        ],
        mesh=mesh,
        compiler_params=pltpu.CompilerParams(needs_layout_passes=False),
        scratch_types=[
            pltpu.VMEM((V,), jnp.float32),
            pltpu.VMEM((CAP,), jnp.float32),
            pltpu.VMEM((CAP,), jnp.int32),
            pltpu.VMEM((CAP,), jnp.int32),
            pltpu.VMEM((OUT,), jnp.float32),
            pltpu.VMEM((OUT,), jnp.int32),
        ],
    )
    return run(flat_logits)


def _tc_final_body(vals_ref, idx_ref, tok_ref, lp_ref):
    v = vals_ref[...]
    ix = idx_ref[...]
    valid = v > NEG
    vs = v / jnp.float32(TEMP)

    sgc = jnp.zeros((B, OUT), jnp.int32)
    for j in range(OUT):
        vj = jnp.broadcast_to(vs[:, j:j + 1], (B, OUT))
        sgc = sgc + (vj > vs).astype(jnp.int32)
    keep_k = valid & (sgc < K)

    vk = jnp.where(keep_k, vs, NEG)
    M = jnp.max(vk, axis=1, keepdims=True)
    e = jnp.where(keep_k, jnp.exp(vk - M), 0.0)
    denom = jnp.sum(e, axis=1, keepdims=True)
    p = e / denom

    cum = jnp.zeros((B, OUT), jnp.float32)
    nb = jnp.zeros((B, OUT), jnp.float32)
    for j in range(OUT):
        vj = jnp.broadcast_to(vs[:, j:j + 1], (B, OUT))
        ij = jnp.broadcast_to(ix[:, j:j + 1], (B, OUT))
        pj = jnp.broadcast_to(p[:, j:j + 1], (B, OUT))
        gt = vj > vs
        eq = vj == vs
        prec = gt | (eq & (ij <= ix))
        sb = gt | (eq & (ij < ix))
        cum = cum + jnp.where(prec, pj, 0.0)
        nb = nb + jnp.where(sb, pj, 0.0)
    remove = (cum > jnp.float32(P_TOP)) & (nb > 0.0)
    keep = keep_k & ~remove

    brow = lax.broadcasted_iota(jnp.int32, (B, OUT), 0)
    flat = brow * V + jnp.where(valid, ix, 0)
    ks0 = jnp.uint32(0)
    ks1 = jnp.uint32(42)
    ks2 = ks0 ^ ks1 ^ jnp.uint32(0x1BD11BDA)
    x0 = jnp.zeros((B, OUT), jnp.uint32) + ks0
    x1 = flat.astype(jnp.uint32) + ks1
    rots = ((13, 15, 26, 6), (17, 29, 16, 24))

    def rounds(x0, x1, rr):
        for r in rr:
            x0 = x0 + x1
            x1 = (x1 << jnp.uint32(r)) | (x1 >> jnp.uint32(32 - r))
            x1 = x1 ^ x0
        return x0, x1

    x0, x1 = rounds(x0, x1, rots[0])
    x0 = x0 + ks1
    x1 = x1 + ks2 + jnp.uint32(1)
    x0, x1 = rounds(x0, x1, rots[1])
    x0 = x0 + ks2
    x1 = x1 + ks0 + jnp.uint32(2)
    x0, x1 = rounds(x0, x1, rots[0])
    x0 = x0 + ks0
    x1 = x1 + ks1 + jnp.uint32(3)
    x0, x1 = rounds(x0, x1, rots[1])
    x0 = x0 + ks1
    x1 = x1 + ks2 + jnp.uint32(4)
    x0, x1 = rounds(x0, x1, rots[0])
    x0 = x0 + ks2
    x1 = x1 + ks0 + jnp.uint32(5)
    bits = x0 ^ x1

    fb = (bits >> jnp.uint32(9)) | jnp.uint32(0x3F800000)
    fl = lax.bitcast_convert_type(fb, jnp.float32) - jnp.float32(1.0)
    tiny = jnp.float32(1.1754943508222875e-38)
    u = jnp.maximum(tiny, fl * (jnp.float32(1.0) - tiny) + tiny)
    g = -jnp.log(-jnp.log(u))

    score = jnp.where(keep, vk + g, NEG)
    smax = jnp.max(score, axis=1, keepdims=True)
    lane = lax.broadcasted_iota(jnp.int32, (B, OUT), 1)
    winlane = jnp.min(jnp.where(score == smax, lane, IMAX), axis=1,
                      keepdims=True)
    iswin = lane == winlane
    tok = jnp.sum(jnp.where(iswin, ix, 0), axis=1, keepdims=True)

    e2 = jnp.where(keep, jnp.exp(vk - M), 0.0)
    den2 = jnp.sum(e2, axis=1, keepdims=True)
    pw = jnp.sum(jnp.where(iswin, e2 / den2, 0.0), axis=1, keepdims=True)

    tok_ref[...] = tok
    lp_ref[...] = jnp.log(pw)


@jax.jit
def _tc_final(cand_vals, cand_idx):
    return pl.pallas_call(
        _tc_final_body,
        out_shape=[
            jax.ShapeDtypeStruct((B, 1), jnp.int32),
            jax.ShapeDtypeStruct((B, 1), jnp.float32),
        ],
    )(cand_vals, cand_idx)


def kernel(logits, top_k):
    del top_k
    cv_flat, ci_flat = _sc_extract(logits.reshape(-1))
    cand_vals = cv_flat.reshape(B, OUT)
    cand_idx = ci_flat.reshape(B, OUT)
    tok, lp = _tc_final(cand_vals, cand_idx)
    return tok.reshape(B), lp


# speedup vs baseline: 1.1235x; 1.1235x over previous
import functools

import jax
import jax.numpy as jnp
from jax import lax
from jax.experimental import pallas as pl
from jax.experimental.pallas import tpu as pltpu
from jax.experimental.pallas import tpu_sc as plsc

B = 128
V = 100000
K = 50
KRAW = 64
OUT = 80
CAP = 2048
UNR = 25
BLK = 10
NW = 32
RPW = B // NW
TEMP = 0.8
P_TOP = 0.95
NEG = float("-inf")
IMAX = 0x7FFFFFFF


def _sc_extract_body(x_hbm, ov_hbm, oi_hbm, data_v, vals_c, idx_c, u_c,
                     stage_v, stage_i, cnt_s):
    sid = lax.axis_index("s")
    wid = sid * 2 + lax.axis_index("c")
    iota16 = lax.iota(jnp.int32, 16)
    neg16 = jnp.full((16,), NEG, jnp.float32)

    lane15 = jnp.full((16,), 15, jnp.int32)

    def slow_scan(off, base, t):
        cv = jnp.zeros((16,), jnp.int32) + jnp.minimum(off, CAP - 16)
        for w in range(BLK):
            b2 = base + w * 16
            v = data_v[pl.ds(b2, 16)]
            m = v >= t
            mi = m.astype(jnp.int32)
            pos = plsc.cumsum(mi)
            tgt = jnp.minimum(cv + pos - mi, CAP - 1)
            plsc.store_scatter(vals_c, [tgt], v, mask=m)
            plsc.store_scatter(idx_c, [tgt], iota16 + b2, mask=m)
            cv = cv + jnp.take(pos, lane15)

    def do_row(r, _):
        row = wid * RPW + r
        pltpu.sync_copy(x_hbm.at[pl.ds(row * V, V)], data_v)

        def top4_body(g, t):
            t1, t2, t3, t4 = t
            x = data_v[pl.ds(g * (16 * UNR), 16)]
            for w in range(1, UNR):
                x = jnp.maximum(x, data_v[pl.ds(g * (16 * UNR) + w * 16, 16)])
            t4 = jnp.maximum(t4, jnp.minimum(x, t3))
            t3 = jnp.maximum(t3, jnp.minimum(x, t2))
            t2 = jnp.maximum(t2, jnp.minimum(x, t1))
            t1 = jnp.maximum(t1, x)
            return t1, t2, t3, t4

        _, _, _, t4 = plsc.parallel_loop(
            0, V // (16 * UNR), unroll=2,
            carry=(neg16, neg16, neg16, neg16))(top4_body)
        t = jnp.min(t4)

        cnt_s[0] = jnp.int32(0)

        def blk_body(b):
            base = b * BLK * 16
            acc = (data_v[pl.ds(base, 16)] >= t).astype(jnp.int32)
            for w in range(1, BLK):
                acc = acc + (data_v[pl.ds(base + w * 16, 16)] >=
                             t).astype(jnp.int32)
            nhit = jnp.sum(acc)

            def slow():
                off = plsc.fetch_and_add(cnt_s, nhit, subcore_id=sid)
                slow_scan(off, base, t)

            pl.when(nhit > 0)(slow)

        plsc.parallel_loop(0, V // (16 * BLK), unroll=2)(blk_body)
        cnt = jnp.minimum(cnt_s[0], jnp.int32(CAP))
        nv = (cnt + 15) // 16

        def mono_body(j, _):
            x = vals_c[pl.ds(j * 16, 16)]
            b = plsc.bitcast(x + jnp.float32(0.0), jnp.int32)
            u = b ^ (lax.shift_right_arithmetic(b, 31) & jnp.int32(IMAX))
            valid = (j * 16 + iota16) < cnt
            u_c[pl.ds(j * 16, 16)] = jnp.where(valid, u,
                                               jnp.int32(-IMAX - 1))
            return 0

        lax.fori_loop(0, nv, mono_body, 0)

        tb = jnp.int32(0)
        for bit in range(31, -1, -1):
            cand = tb | (jnp.int32(1) << bit)
            probe = cand ^ jnp.int32(-IMAX - 1)

            def cnt_body(j, c, probe=probe):
                u = u_c[pl.ds(j * 16, 16)]
                return c + jnp.sum((u >= probe).astype(jnp.int32))

            c = lax.fori_loop(0, nv, cnt_body, jnp.int32(0))
            tb = jnp.where(c >= KRAW, cand, tb)
        t64 = tb ^ jnp.int32(-IMAX - 1)

        for jj in range(OUT // 16):
            stage_v[pl.ds(jj * 16, 16)] = neg16
            stage_i[pl.ds(jj * 16, 16)] = jnp.full((16,), IMAX, jnp.int32)

        def fcompact(j, oc):
            x = vals_c[pl.ds(j * 16, 16)]
            ix = idx_c[pl.ds(j * 16, 16)]
            u = u_c[pl.ds(j * 16, 16)]
            m = u >= t64
            s = jnp.sum(m.astype(jnp.int32))

            def do_store():
                plsc.store_compressed(stage_v.at[pl.ds(oc, 16)], x, mask=m)
                plsc.store_compressed(stage_i.at[pl.ds(oc, 16)], ix, mask=m)

            pl.when(oc + s <= OUT)(do_store)
            return oc + s

        lax.fori_loop(0, nv, fcompact, jnp.int32(0))

        pltpu.sync_copy(stage_v, ov_hbm.at[pl.ds(row * OUT, OUT)])
        pltpu.sync_copy(stage_i, oi_hbm.at[pl.ds(row * OUT, OUT)])
        return 0

    lax.fori_loop(0, RPW, do_row, 0)


@jax.jit
def _sc_extract(flat_logits):
    mesh = plsc.VectorSubcoreMesh(core_axis_name="c", subcore_axis_name="s")
    run = pl.kernel(
        _sc_extract_body,
        out_type=[
            jax.ShapeDtypeStruct((B * OUT,), jnp.float32),
            jax.ShapeDtypeStruct((B * OUT,), jnp.int32),
        ],
        mesh=mesh,
        compiler_params=pltpu.CompilerParams(needs_layout_passes=False),
        scratch_types=[
            pltpu.VMEM((V,), jnp.float32),
            pltpu.VMEM((CAP,), jnp.float32),
            pltpu.VMEM((CAP,), jnp.int32),
            pltpu.VMEM((CAP,), jnp.int32),
            pltpu.VMEM((OUT,), jnp.float32),
            pltpu.VMEM((OUT,), jnp.int32),
            pltpu.SMEM((1,), jnp.int32),
        ],
    )
    return run(flat_logits)


def _tc_final_body(vals_ref, idx_ref, tok_ref, lp_ref):
    v = vals_ref[...]
    ix = idx_ref[...]
    valid = v > NEG
    vs = v / jnp.float32(TEMP)

    sgc = jnp.zeros((B, OUT), jnp.int32)
    for j in range(OUT):
        vj = jnp.broadcast_to(vs[:, j:j + 1], (B, OUT))
        sgc = sgc + (vj > vs).astype(jnp.int32)
    keep_k = valid & (sgc < K)

    vk = jnp.where(keep_k, vs, NEG)
    M = jnp.max(vk, axis=1, keepdims=True)
    e = jnp.where(keep_k, jnp.exp(vk - M), 0.0)
    denom = jnp.sum(e, axis=1, keepdims=True)
    p = e / denom

    cum = jnp.zeros((B, OUT), jnp.float32)
    nb = jnp.zeros((B, OUT), jnp.float32)
    for j in range(OUT):
        vj = jnp.broadcast_to(vs[:, j:j + 1], (B, OUT))
        ij = jnp.broadcast_to(ix[:, j:j + 1], (B, OUT))
        pj = jnp.broadcast_to(p[:, j:j + 1], (B, OUT))
        gt = vj > vs
        eq = vj == vs
        prec = gt | (eq & (ij <= ix))
        sb = gt | (eq & (ij < ix))
        cum = cum + jnp.where(prec, pj, 0.0)
        nb = nb + jnp.where(sb, pj, 0.0)
    remove = (cum > jnp.float32(P_TOP)) & (nb > 0.0)
    keep = keep_k & ~remove

    brow = lax.broadcasted_iota(jnp.int32, (B, OUT), 0)
    flat = brow * V + jnp.where(valid, ix, 0)
    ks0 = jnp.uint32(0)
    ks1 = jnp.uint32(42)
    ks2 = ks0 ^ ks1 ^ jnp.uint32(0x1BD11BDA)
    x0 = jnp.zeros((B, OUT), jnp.uint32) + ks0
    x1 = flat.astype(jnp.uint32) + ks1
    rots = ((13, 15, 26, 6), (17, 29, 16, 24))

    def rounds(x0, x1, rr):
        for r in rr:
            x0 = x0 + x1
            x1 = (x1 << jnp.uint32(r)) | (x1 >> jnp.uint32(32 - r))
            x1 = x1 ^ x0
        return x0, x1

    x0, x1 = rounds(x0, x1, rots[0])
    x0 = x0 + ks1
    x1 = x1 + ks2 + jnp.uint32(1)
    x0, x1 = rounds(x0, x1, rots[1])
    x0 = x0 + ks2
    x1 = x1 + ks0 + jnp.uint32(2)
    x0, x1 = rounds(x0, x1, rots[0])
    x0 = x0 + ks0
    x1 = x1 + ks1 + jnp.uint32(3)
    x0, x1 = rounds(x0, x1, rots[1])
    x0 = x0 + ks1
    x1 = x1 + ks2 + jnp.uint32(4)
    x0, x1 = rounds(x0, x1, rots[0])
    x0 = x0 + ks2
    x1 = x1 + ks0 + jnp.uint32(5)
    bits = x0 ^ x1

    fb = (bits >> jnp.uint32(9)) | jnp.uint32(0x3F800000)
    fl = lax.bitcast_convert_type(fb, jnp.float32) - jnp.float32(1.0)
    tiny = jnp.float32(1.1754943508222875e-38)
    u = jnp.maximum(tiny, fl * (jnp.float32(1.0) - tiny) + tiny)
    g = -jnp.log(-jnp.log(u))

    score = jnp.where(keep, vk + g, NEG)
    smax = jnp.max(score, axis=1, keepdims=True)
    lane = lax.broadcasted_iota(jnp.int32, (B, OUT), 1)
    winlane = jnp.min(jnp.where(score == smax, lane, IMAX), axis=1,
                      keepdims=True)
    iswin = lane == winlane
    tok = jnp.sum(jnp.where(iswin, ix, 0), axis=1, keepdims=True)

    e2 = jnp.where(keep, jnp.exp(vk - M), 0.0)
    den2 = jnp.sum(e2, axis=1, keepdims=True)
    pw = jnp.sum(jnp.where(iswin, e2 / den2, 0.0), axis=1, keepdims=True)

    tok_ref[...] = tok
    lp_ref[...] = jnp.log(pw)


@jax.jit
def _tc_final(cand_vals, cand_idx):
    return pl.pallas_call(
        _tc_final_body,
        out_shape=[
            jax.ShapeDtypeStruct((B, 1), jnp.int32),
            jax.ShapeDtypeStruct((B, 1), jnp.float32),
        ],
    )(cand_vals, cand_idx)


def kernel(logits, top_k):
    del top_k
    cv_flat, ci_flat = _sc_extract(logits.reshape(-1))
    cand_vals = cv_flat.reshape(B, OUT)
    cand_idx = ci_flat.reshape(B, OUT)
    tok, lp = _tc_final(cand_vals, cand_idx)
    return tok.reshape(B), lp


# speedup vs baseline: 1.1666x; 1.0383x over previous
import functools

import jax
import jax.numpy as jnp
from jax import lax
from jax.experimental import pallas as pl
from jax.experimental.pallas import tpu as pltpu
from jax.experimental.pallas import tpu_sc as plsc

B = 128
V = 100000
K = 50
KRAW = 64
OUT = 80
CAP = 2048
UNR = 25
BLK = 10
NW = 32
RPW = B // NW
TEMP = 0.8
P_TOP = 0.95
NEG = float("-inf")
IMAX = 0x7FFFFFFF


def _sc_extract_body(x_hbm, ov_hbm, oi_hbm, data_v, vals_c, idx_c, u_c,
                     stage_v, stage_i):
    wid = lax.axis_index("s") * 2 + lax.axis_index("c")
    iota16 = lax.iota(jnp.int32, 16)
    neg16 = jnp.full((16,), NEG, jnp.float32)

    lane15 = jnp.full((16,), 15, jnp.int32)

    def slow_scan(off, base, t):
        cv = jnp.zeros((16,), jnp.int32) + jnp.minimum(off, CAP - 16)
        for w in range(BLK):
            b2 = base + w * 16
            v = data_v[pl.ds(b2, 16)]
            m = v >= t
            mi = m.astype(jnp.int32)
            pos = plsc.cumsum(mi)
            tgt = jnp.minimum(cv + pos - mi, CAP - 1)
            plsc.store_scatter(vals_c, [tgt], v, mask=m)
            plsc.store_scatter(idx_c, [tgt], iota16 + b2, mask=m)
            cv = cv + jnp.take(pos, lane15)

    def do_row(r, _):
        row = wid * RPW + r
        pltpu.sync_copy(x_hbm.at[pl.ds(row * V, V)], data_v)

        def top4_body(g, t):
            t1, t2, t3, t4 = t
            x = data_v[pl.ds(g * (16 * UNR), 16)]
            for w in range(1, UNR):
                x = jnp.maximum(x, data_v[pl.ds(g * (16 * UNR) + w * 16, 16)])
            t4 = jnp.maximum(t4, jnp.minimum(x, t3))
            t3 = jnp.maximum(t3, jnp.minimum(x, t2))
            t2 = jnp.maximum(t2, jnp.minimum(x, t1))
            t1 = jnp.maximum(t1, x)
            return t1, t2, t3, t4

        _, _, _, t4 = plsc.parallel_loop(
            0, V // (16 * UNR), unroll=2,
            carry=(neg16, neg16, neg16, neg16))(top4_body)
        t = jnp.min(t4)

        def blk_body(b, carry):
            cnt, psum, pbase = carry

            def run_slow(c, pbase=pbase, psum=psum):
                slow_scan(c, pbase, t)
                return c + psum

            cnt = lax.cond(psum > 0, run_slow, lambda c: c, cnt)
            base = b * BLK * 16
            acc = (data_v[pl.ds(base, 16)] >= t).astype(jnp.int32)
            for w in range(1, BLK):
                acc = acc + (data_v[pl.ds(base + w * 16, 16)] >=
                             t).astype(jnp.int32)
            nhit = jnp.sum(acc)
            return cnt, nhit, base

        cnt, psum, pbase = lax.fori_loop(
            0, V // (16 * BLK), blk_body,
            (jnp.int32(0), jnp.int32(0), jnp.int32(0)))

        def run_slow_tail(c):
            slow_scan(c, pbase, t)
            return c + psum

        cnt = lax.cond(psum > 0, run_slow_tail, lambda c: c, cnt)
        cnt = jnp.minimum(cnt, jnp.int32(CAP))
        nv = (cnt + 15) // 16

        def mono_body(j, _):
            x = vals_c[pl.ds(j * 16, 16)]
            b = plsc.bitcast(x + jnp.float32(0.0), jnp.int32)
            u = b ^ (lax.shift_right_arithmetic(b, 31) & jnp.int32(IMAX))
            valid = (j * 16 + iota16) < cnt
            u_c[pl.ds(j * 16, 16)] = jnp.where(valid, u,
                                               jnp.int32(-IMAX - 1))
            return 0

        lax.fori_loop(0, nv, mono_body, 0)

        tb = jnp.int32(0)
        for bit in range(31, -1, -1):
            cand = tb | (jnp.int32(1) << bit)
            probe = cand ^ jnp.int32(-IMAX - 1)

            def cnt_body(j, c, probe=probe):
                u = u_c[pl.ds(j * 16, 16)]
                return c + jnp.sum((u >= probe).astype(jnp.int32))

            c = lax.fori_loop(0, nv, cnt_body, jnp.int32(0))
            tb = jnp.where(c >= KRAW, cand, tb)
        t64 = tb ^ jnp.int32(-IMAX - 1)

        for jj in range(OUT // 16):
            stage_v[pl.ds(jj * 16, 16)] = neg16
            stage_i[pl.ds(jj * 16, 16)] = jnp.full((16,), IMAX, jnp.int32)

        def fcompact(j, oc):
            x = vals_c[pl.ds(j * 16, 16)]
            ix = idx_c[pl.ds(j * 16, 16)]
            u = u_c[pl.ds(j * 16, 16)]
            m = u >= t64
            s = jnp.sum(m.astype(jnp.int32))

            def do_store():
                plsc.store_compressed(stage_v.at[pl.ds(oc, 16)], x, mask=m)
                plsc.store_compressed(stage_i.at[pl.ds(oc, 16)], ix, mask=m)

            pl.when(oc + s <= OUT)(do_store)
            return oc + s

        lax.fori_loop(0, nv, fcompact, jnp.int32(0))

        pltpu.sync_copy(stage_v, ov_hbm.at[pl.ds(row * OUT, OUT)])
        pltpu.sync_copy(stage_i, oi_hbm.at[pl.ds(row * OUT, OUT)])
        return 0

    lax.fori_loop(0, RPW, do_row, 0)


@jax.jit
def _sc_extract(flat_logits):
    mesh = plsc.VectorSubcoreMesh(core_axis_name="c", subcore_axis_name="s")
    run = pl.kernel(
        _sc_extract_body,
        out_type=[
            jax.ShapeDtypeStruct((B * OUT,), jnp.float32),
            jax.ShapeDtypeStruct((B * OUT,), jnp.int32),
        ],
        mesh=mesh,
        compiler_params=pltpu.CompilerParams(needs_layout_passes=False),
        scratch_types=[
            pltpu.VMEM((V,), jnp.float32),
            pltpu.VMEM((CAP,), jnp.float32),
            pltpu.VMEM((CAP,), jnp.int32),
            pltpu.VMEM((CAP,), jnp.int32),
            pltpu.VMEM((OUT,), jnp.float32),
            pltpu.VMEM((OUT,), jnp.int32),
        ],
    )
    return run(flat_logits)


def _tc_final_body(vals_ref, idx_ref, tok_ref, lp_ref):
    v = vals_ref[...]
    ix = idx_ref[...]
    valid = v > NEG
    vs = v / jnp.float32(TEMP)

    sgc = jnp.zeros((B, OUT), jnp.int32)
    for j in range(OUT):
        vj = jnp.broadcast_to(vs[:, j:j + 1], (B, OUT))
        sgc = sgc + (vj > vs).astype(jnp.int32)
    keep_k = valid & (sgc < K)

    vk = jnp.where(keep_k, vs, NEG)
    M = jnp.max(vk, axis=1, keepdims=True)
    e = jnp.where(keep_k, jnp.exp(vk - M), 0.0)
    denom = jnp.sum(e, axis=1, keepdims=True)
    p = e / denom

    cum = jnp.zeros((B, OUT), jnp.float32)
    nb = jnp.zeros((B, OUT), jnp.float32)
    for j in range(OUT):
        vj = jnp.broadcast_to(vs[:, j:j + 1], (B, OUT))
        ij = jnp.broadcast_to(ix[:, j:j + 1], (B, OUT))
        pj = jnp.broadcast_to(p[:, j:j + 1], (B, OUT))
        gt = vj > vs
        eq = vj == vs
        prec = gt | (eq & (ij <= ix))
        sb = gt | (eq & (ij < ix))
        cum = cum + jnp.where(prec, pj, 0.0)
        nb = nb + jnp.where(sb, pj, 0.0)
    remove = (cum > jnp.float32(P_TOP)) & (nb > 0.0)
    keep = keep_k & ~remove

    brow = lax.broadcasted_iota(jnp.int32, (B, OUT), 0)
    flat = brow * V + jnp.where(valid, ix, 0)
    ks0 = jnp.uint32(0)
    ks1 = jnp.uint32(42)
    ks2 = ks0 ^ ks1 ^ jnp.uint32(0x1BD11BDA)
    x0 = jnp.zeros((B, OUT), jnp.uint32) + ks0
    x1 = flat.astype(jnp.uint32) + ks1
    rots = ((13, 15, 26, 6), (17, 29, 16, 24))

    def rounds(x0, x1, rr):
        for r in rr:
            x0 = x0 + x1
            x1 = (x1 << jnp.uint32(r)) | (x1 >> jnp.uint32(32 - r))
            x1 = x1 ^ x0
        return x0, x1

    x0, x1 = rounds(x0, x1, rots[0])
    x0 = x0 + ks1
    x1 = x1 + ks2 + jnp.uint32(1)
    x0, x1 = rounds(x0, x1, rots[1])
    x0 = x0 + ks2
    x1 = x1 + ks0 + jnp.uint32(2)
    x0, x1 = rounds(x0, x1, rots[0])
    x0 = x0 + ks0
    x1 = x1 + ks1 + jnp.uint32(3)
    x0, x1 = rounds(x0, x1, rots[1])
    x0 = x0 + ks1
    x1 = x1 + ks2 + jnp.uint32(4)
    x0, x1 = rounds(x0, x1, rots[0])
    x0 = x0 + ks2
    x1 = x1 + ks0 + jnp.uint32(5)
    bits = x0 ^ x1

    fb = (bits >> jnp.uint32(9)) | jnp.uint32(0x3F800000)
    fl = lax.bitcast_convert_type(fb, jnp.float32) - jnp.float32(1.0)
    tiny = jnp.float32(1.1754943508222875e-38)
    u = jnp.maximum(tiny, fl * (jnp.float32(1.0) - tiny) + tiny)
    g = -jnp.log(-jnp.log(u))

    score = jnp.where(keep, vk + g, NEG)
    smax = jnp.max(score, axis=1, keepdims=True)
    lane = lax.broadcasted_iota(jnp.int32, (B, OUT), 1)
    winlane = jnp.min(jnp.where(score == smax, lane, IMAX), axis=1,
                      keepdims=True)
    iswin = lane == winlane
    tok = jnp.sum(jnp.where(iswin, ix, 0), axis=1, keepdims=True)

    e2 = jnp.where(keep, jnp.exp(vk - M), 0.0)
    den2 = jnp.sum(e2, axis=1, keepdims=True)
    pw = jnp.sum(jnp.where(iswin, e2 / den2, 0.0), axis=1, keepdims=True)

    tok_ref[...] = tok
    lp_ref[...] = jnp.log(pw)


@jax.jit
def _tc_final(cand_vals, cand_idx):
    return pl.pallas_call(
        _tc_final_body,
        out_shape=[
            jax.ShapeDtypeStruct((B, 1), jnp.int32),
            jax.ShapeDtypeStruct((B, 1), jnp.float32),
        ],
    )(cand_vals, cand_idx)


def kernel(logits, top_k):
    del top_k
    cv_flat, ci_flat = _sc_extract(logits.reshape(-1))
    cand_vals = cv_flat.reshape(B, OUT)
    cand_idx = ci_flat.reshape(B, OUT)
    tok, lp = _tc_final(cand_vals, cand_idx)
    return tok.reshape(B), lp


# speedup vs baseline: 1.1666x; 1.0000x over previous
import functools

import jax
import jax.numpy as jnp
from jax import lax
from jax.experimental import pallas as pl
from jax.experimental.pallas import tpu as pltpu
from jax.experimental.pallas import tpu_sc as plsc

B = 128
V = 100000
K = 50
KRAW = 64
OUT = 80
CAP = 2048
UNR = 25
BLK = 10
NW = 32
RPW = B // NW
TEMP = 0.8
P_TOP = 0.95
NEG = float("-inf")
IMAX = 0x7FFFFFFF


def _sc_extract_body(x_hbm, ov_hbm, oi_hbm, data_v, vals_c, idx_c, u_c,
                     stage_v, stage_i):
    wid = lax.axis_index("s") * 2 + lax.axis_index("c")
    iota16 = lax.iota(jnp.int32, 16)
    neg16 = jnp.full((16,), NEG, jnp.float32)

    lane15 = jnp.full((16,), 15, jnp.int32)

    def slow_scan(off, base, t):
        cv = jnp.zeros((16,), jnp.int32) + jnp.minimum(off, CAP - 16)
        for w in range(BLK):
            b2 = base + w * 16
            v = data_v[pl.ds(b2, 16)]
            m = v >= t
            mi = m.astype(jnp.int32)
            pos = plsc.cumsum(mi)
            tgt = jnp.minimum(cv + pos - mi, CAP - 1)
            plsc.store_scatter(vals_c, [tgt], v, mask=m)
            plsc.store_scatter(idx_c, [tgt], iota16 + b2, mask=m)
            cv = cv + jnp.take(pos, lane15)

    def do_row(r, _):
        row = wid * RPW + r
        pltpu.sync_copy(x_hbm.at[pl.ds(row * V, V)], data_v)

        def top4_body(g, t):
            t1, t2, t3, t4 = t
            x = data_v[pl.ds(g * (16 * UNR), 16)]
            for w in range(1, UNR):
                x = jnp.maximum(x, data_v[pl.ds(g * (16 * UNR) + w * 16, 16)])
            t4 = jnp.maximum(t4, jnp.minimum(x, t3))
            t3 = jnp.maximum(t3, jnp.minimum(x, t2))
            t2 = jnp.maximum(t2, jnp.minimum(x, t1))
            t1 = jnp.maximum(t1, x)
            return t1, t2, t3, t4

        _, _, _, t4 = plsc.parallel_loop(
            0, V // (16 * UNR), unroll=2,
            carry=(neg16, neg16, neg16, neg16))(top4_body)
        t = jnp.min(t4)

        def blk_body(b, carry):
            cnt, psum, pbase = carry

            def run_slow(c, pbase=pbase, psum=psum):
                slow_scan(c, pbase, t)
                return c + psum

            cnt = lax.cond(psum > 0, run_slow, lambda c: c, cnt)
            base = b * BLK * 16
            acc = (data_v[pl.ds(base, 16)] >= t).astype(jnp.int32)
            for w in range(1, BLK):
                acc = acc + (data_v[pl.ds(base + w * 16, 16)] >=
                             t).astype(jnp.int32)
            nhit = jnp.sum(acc)
            return cnt, nhit, base

        cnt, psum, pbase = lax.fori_loop(
            0, V // (16 * BLK), blk_body,
            (jnp.int32(0), jnp.int32(0), jnp.int32(0)))

        def run_slow_tail(c):
            slow_scan(c, pbase, t)
            return c + psum

        cnt = lax.cond(psum > 0, run_slow_tail, lambda c: c, cnt)
        cnt = jnp.minimum(cnt, jnp.int32(CAP))
        nv = (cnt + 15) // 16

        def mono_body(j, _):
            x = vals_c[pl.ds(j * 16, 16)]
            b = plsc.bitcast(x + jnp.float32(0.0), jnp.int32)
            u = b ^ (lax.shift_right_arithmetic(b, 31) & jnp.int32(IMAX))
            valid = (j * 16 + iota16) < cnt
            u_c[pl.ds(j * 16, 16)] = jnp.where(valid, u,
                                               jnp.int32(-IMAX - 1))
            return 0

        lax.fori_loop(0, nv, mono_body, 0)

        tb = jnp.int32(0)
        for bit in range(31, -1, -1):
            cand = tb | (jnp.int32(1) << bit)
            probe = cand ^ jnp.int32(-IMAX - 1)

            def cnt_body(j, c, probe=probe):
                u = u_c[pl.ds(j * 16, 16)]
                return c + jnp.sum((u >= probe).astype(jnp.int32))

            c = lax.fori_loop(0, nv, cnt_body, jnp.int32(0))
            tb = jnp.where(c >= KRAW, cand, tb)
        t64 = tb ^ jnp.int32(-IMAX - 1)

        for jj in range(OUT // 16):
            stage_v[pl.ds(jj * 16, 16)] = neg16
            stage_i[pl.ds(jj * 16, 16)] = jnp.full((16,), IMAX, jnp.int32)

        def fcompact(j, oc):
            x = vals_c[pl.ds(j * 16, 16)]
            ix = idx_c[pl.ds(j * 16, 16)]
            u = u_c[pl.ds(j * 16, 16)]
            m = u >= t64
            s = jnp.sum(m.astype(jnp.int32))

            def do_store():
                plsc.store_compressed(stage_v.at[pl.ds(oc, 16)], x, mask=m)
                plsc.store_compressed(stage_i.at[pl.ds(oc, 16)], ix, mask=m)

            pl.when(oc + s <= OUT)(do_store)
            return oc + s

        lax.fori_loop(0, nv, fcompact, jnp.int32(0))

        pltpu.sync_copy(stage_v, ov_hbm.at[pl.ds(row * OUT, OUT)])
        pltpu.sync_copy(stage_i, oi_hbm.at[pl.ds(row * OUT, OUT)])
        return 0

    lax.fori_loop(0, RPW, do_row, 0)


@jax.jit
def _sc_extract(flat_logits):
    mesh = plsc.VectorSubcoreMesh(core_axis_name="c", subcore_axis_name="s")
    run = pl.kernel(
        _sc_extract_body,
        out_type=[
            jax.ShapeDtypeStruct((B * OUT,), jnp.float32),
            jax.ShapeDtypeStruct((B * OUT,), jnp.int32),
        ],
        mesh=mesh,
        compiler_params=pltpu.CompilerParams(needs_layout_passes=False,
                                             use_tc_tiling_on_sc=True),
        scratch_types=[
            pltpu.VMEM((V,), jnp.float32),
            pltpu.VMEM((CAP,), jnp.float32),
            pltpu.VMEM((CAP,), jnp.int32),
            pltpu.VMEM((CAP,), jnp.int32),
            pltpu.VMEM((OUT,), jnp.float32),
            pltpu.VMEM((OUT,), jnp.int32),
        ],
    )
    return run(flat_logits)


def _tc_final_body(vals_ref, idx_ref, tok_ref, lp_ref):
    v = vals_ref[...]
    ix = idx_ref[...]
    valid = v > NEG
    vs = v / jnp.float32(TEMP)

    sgc = jnp.zeros((B, OUT), jnp.int32)
    for j in range(OUT):
        vj = jnp.broadcast_to(vs[:, j:j + 1], (B, OUT))
        sgc = sgc + (vj > vs).astype(jnp.int32)
    keep_k = valid & (sgc < K)

    vk = jnp.where(keep_k, vs, NEG)
    M = jnp.max(vk, axis=1, keepdims=True)
    e = jnp.where(keep_k, jnp.exp(vk - M), 0.0)
    denom = jnp.sum(e, axis=1, keepdims=True)
    p = e / denom

    cum = jnp.zeros((B, OUT), jnp.float32)
    nb = jnp.zeros((B, OUT), jnp.float32)
    for j in range(OUT):
        vj = jnp.broadcast_to(vs[:, j:j + 1], (B, OUT))
        ij = jnp.broadcast_to(ix[:, j:j + 1], (B, OUT))
        pj = jnp.broadcast_to(p[:, j:j + 1], (B, OUT))
        gt = vj > vs
        eq = vj == vs
        prec = gt | (eq & (ij <= ix))
        sb = gt | (eq & (ij < ix))
        cum = cum + jnp.where(prec, pj, 0.0)
        nb = nb + jnp.where(sb, pj, 0.0)
    remove = (cum > jnp.float32(P_TOP)) & (nb > 0.0)
    keep = keep_k & ~remove

    brow = lax.broadcasted_iota(jnp.int32, (B, OUT), 0)
    flat = brow * V + jnp.where(valid, ix, 0)
    ks0 = jnp.uint32(0)
    ks1 = jnp.uint32(42)
    ks2 = ks0 ^ ks1 ^ jnp.uint32(0x1BD11BDA)
    x0 = jnp.zeros((B, OUT), jnp.uint32) + ks0
    x1 = flat.astype(jnp.uint32) + ks1
    rots = ((13, 15, 26, 6), (17, 29, 16, 24))

    def rounds(x0, x1, rr):
        for r in rr:
            x0 = x0 + x1
            x1 = (x1 << jnp.uint32(r)) | (x1 >> jnp.uint32(32 - r))
            x1 = x1 ^ x0
        return x0, x1

    x0, x1 = rounds(x0, x1, rots[0])
    x0 = x0 + ks1
    x1 = x1 + ks2 + jnp.uint32(1)
    x0, x1 = rounds(x0, x1, rots[1])
    x0 = x0 + ks2
    x1 = x1 + ks0 + jnp.uint32(2)
    x0, x1 = rounds(x0, x1, rots[0])
    x0 = x0 + ks0
    x1 = x1 + ks1 + jnp.uint32(3)
    x0, x1 = rounds(x0, x1, rots[1])
    x0 = x0 + ks1
    x1 = x1 + ks2 + jnp.uint32(4)
    x0, x1 = rounds(x0, x1, rots[0])
    x0 = x0 + ks2
    x1 = x1 + ks0 + jnp.uint32(5)
    bits = x0 ^ x1

    fb = (bits >> jnp.uint32(9)) | jnp.uint32(0x3F800000)
    fl = lax.bitcast_convert_type(fb, jnp.float32) - jnp.float32(1.0)
    tiny = jnp.float32(1.1754943508222875e-38)
    u = jnp.maximum(tiny, fl * (jnp.float32(1.0) - tiny) + tiny)
    g = -jnp.log(-jnp.log(u))

    score = jnp.where(keep, vk + g, NEG)
    smax = jnp.max(score, axis=1, keepdims=True)
    lane = lax.broadcasted_iota(jnp.int32, (B, OUT), 1)
    winlane = jnp.min(jnp.where(score == smax, lane, IMAX), axis=1,
                      keepdims=True)
    iswin = lane == winlane
    tok = jnp.sum(jnp.where(iswin, ix, 0), axis=1, keepdims=True)

    e2 = jnp.where(keep, jnp.exp(vk - M), 0.0)
    den2 = jnp.sum(e2, axis=1, keepdims=True)
    pw = jnp.sum(jnp.where(iswin, e2 / den2, 0.0), axis=1, keepdims=True)

    tok_ref[...] = tok
    lp_ref[...] = jnp.log(pw)


@jax.jit
def _tc_final(cand_vals, cand_idx):
    return pl.pallas_call(
        _tc_final_body,
        out_shape=[
            jax.ShapeDtypeStruct((B, 1), jnp.int32),
            jax.ShapeDtypeStruct((B, 1), jnp.float32),
        ],
    )(cand_vals, cand_idx)


def kernel(logits, top_k):
    del top_k
    cv_flat, ci_flat = _sc_extract(logits.reshape(-1))
    cand_vals = cv_flat.reshape(B, OUT)
    cand_idx = ci_flat.reshape(B, OUT)
    tok, lp = _tc_final(cand_vals, cand_idx)
    return tok.reshape(B), lp
